# baseline (speedup 1.0000x reference)
import jax
import jax.numpy as jnp
from jax import lax
from jax.experimental import pallas as pl
from jax.experimental.pallas import tpu as pltpu
from jax.experimental.pallas import tpu_sc as plsc

N = 10000
D = 128
HD = 64
NP = 10240
RPT = 640
RCH = 64
ZCH = 32
NBUF = 3
HBF = 3
ECH = 128
CPB = 16
NBLK = 20
EPT = NBLK * CPB * ECH
E2P = 16 * EPT
IPT = EPT // ECH
E2 = 2 * 320000
THETA_K = (-0.5, 0.25, -0.125)
DGR = NP // HD


def _rsqrt(x):
    xi = plsc.bitcast(x, jnp.int32)
    y = plsc.bitcast(jnp.int32(0x5F3759DF) - (xi >> 1), jnp.float32)
    for _ in range(3):
        y = y * (1.5 - 0.5 * x * y * y)
    return y


def _splat(vec_ref, i):
    return plsc.load_gather(vec_ref, [jnp.full((16,), i, jnp.int32)])


def _sc_body(src_hbm, dst_hbm, feat_hbm, out_hbm, g_hbm,
             g_sh, agg_sh,
             rows_v, zbuf_v, gbuf_v, abuf_v, hbuf_v,
             sidx_v, didx_v, dinv2_v, dsqrt_v, gsem, hsem, ssem):
    c = lax.axis_index("c")
    s = lax.axis_index("s")
    r0 = s * RPT
    zeros16 = jnp.zeros((16,), jnp.float32)
    ones16 = jnp.ones((16,), jnp.float32)

    def zero_ba(i, carry):
        for q in range(HD // 16):
            gbuf_v[i, pl.ds(q * 16, 16)] = zeros16
            abuf_v[i, pl.ds(q * 16, 16)] = zeros16
            hbuf_v[i, pl.ds(q * 16, 16)] = zeros16
        return carry
    lax.fori_loop(0, RCH, zero_ba, 0)

    def deg_blk(blk, carry):
        irow = s * IPT + blk * CPB
        pltpu.sync_copy(dst_hbm.at[pl.ds(irow, CPB), :], didx_v)
        def deg_row(j, carry2):
            def deg_inner(i, carry3):
                idx = didx_v[j, pl.ds(i * 16, 16)]
                row = idx >> 6
                col = idx & 63
                plsc.addupdate_scatter(gbuf_v, [row, col], ones16,
                                       mask=row < RCH)
                plsc.addupdate_scatter(abuf_v, [row - RCH, col], ones16,
                                       mask=(row >= RCH) & (row < 2 * RCH))
                plsc.addupdate_scatter(hbuf_v, [row - 2 * RCH, col], ones16,
                                       mask=row >= 2 * RCH)
                return carry3
            return lax.fori_loop(0, ECH // 16, deg_inner, carry2)
        return lax.fori_loop(0, CPB, deg_row, carry)
    lax.fori_loop(0, NBLK, deg_blk, 0)

    pltpu.sync_copy(gbuf_v, agg_sh.at[pl.ds(r0, RCH), :])
    pltpu.sync_copy(abuf_v, agg_sh.at[pl.ds(r0 + RCH, RCH), :])
    pltpu.sync_copy(hbuf_v.at[pl.ds(0, DGR - 2 * RCH), :],
                    agg_sh.at[pl.ds(r0 + 2 * RCH, DGR - 2 * RCH), :])
    plsc.subcore_barrier()

    def zero_g10(i, carry):
        for q in range(HD // 16):
            gbuf_v[i, pl.ds(q * 16, 16)] = zeros16
        return carry
    lax.fori_loop(0, RPT // HD, zero_g10, 0)
    def deg_reduce(t, carry):
        pltpu.sync_copy(agg_sh.at[pl.ds(t * RPT + s * 10, RPT // HD), :],
                        abuf_v.at[pl.ds(0, RPT // HD), :])
        def acc_row(i, carry2):
            for q in range(HD // 16):
                sl = pl.ds(q * 16, 16)
                gbuf_v[i, sl] = gbuf_v[i, sl] + abuf_v[i, sl]
            return carry2
        return lax.fori_loop(0, RPT // HD, acc_row, carry)
    lax.fori_loop(0, 16, deg_reduce, 0)

    def dinv_chunk(j, carry):
        sl = pl.ds((j % 4) * 16, 16)
        x = jnp.maximum(gbuf_v[j // 4, sl], 1.0)
        dv = _rsqrt(x)
        dinv2_v[pl.ds(j * 16, 16)] = dv * dv
        dsqrt_v[pl.ds(j * 16, 16)] = x * dv
        return carry
    lax.fori_loop(0, RPT // 16, dinv_chunk, 0)
    plsc.subcore_barrier()

    def zero_z(i, carry):
        for q in range(HD // 16):
            zbuf_v[i, pl.ds(q * 16, 16)] = zeros16
        return carry
    lax.fori_loop(0, ZCH, zero_z, 0)
    for ch in range(RPT // ZCH):
        pltpu.sync_copy(zbuf_v, agg_sh.at[pl.ds(r0 + ch * ZCH, ZCH), :])
    pltpu.sync_copy(zbuf_v.at[pl.ds(0, 15), :],
                    g_sh.at[pl.ds(N + s * 15, 15), :])

    for ch in range(RPT // RCH):
        rbase = r0 + ch * RCH
        pltpu.sync_copy(feat_hbm.at[c, pl.ds(rbase, RCH), :], gbuf_v)
        pltpu.sync_copy(gbuf_v, out_hbm.at[c, pl.ds(rbase, RCH), :])
        def init_row(r, carry):
            dv = _splat(dinv2_v, ch * RCH + r) * _splat(dsqrt_v, ch * RCH + r)
            for q in range(HD // 16):
                sl = pl.ds(q * 16, 16)
                gbuf_v[r, sl] = gbuf_v[r, sl] * dv
            return carry
        lax.fori_loop(0, RCH, init_row, 0)
        pltpu.sync_copy(gbuf_v, g_sh.at[pl.ds(rbase, RCH), :])
        pltpu.sync_copy(gbuf_v, g_hbm.at[c, pl.ds(rbase, RCH), :])

    plsc.subcore_barrier()

    for k, theta in enumerate(THETA_K):
        last = k == len(THETA_K) - 1

        def gsrc(j):
            if j % 8 < HBF:
                return g_hbm.at[c].at[sidx_v.at[j]], hsem
            return g_sh.at[sidx_v.at[j]], gsem

        def edge_blk(blk, carry):
            irow = s * IPT + blk * CPB
            pltpu.sync_copy(src_hbm.at[pl.ds(irow, CPB), :], sidx_v)
            pltpu.sync_copy(dst_hbm.at[pl.ds(irow, CPB), :], didx_v)
            for j in range(NBUF - 1):
                ref, sem = gsrc(j)
                pltpu.async_copy(ref, rows_v.at[j], sem)
            for j in range(CPB):
                b = j % NBUF
                if j + NBUF - 1 < CPB:
                    if j >= 1:
                        pltpu.make_async_copy(
                            rows_v.at[(j - 1) % NBUF],
                            agg_sh.at[didx_v.at[j - 1]], ssem).wait()
                    ref, sem = gsrc(j + NBUF - 1)
                    pltpu.async_copy(ref, rows_v.at[(j + NBUF - 1) % NBUF],
                                     sem)
                ref, sem = gsrc(j)
                pltpu.make_async_copy(ref, rows_v.at[b], sem).wait()
                pltpu.async_copy(rows_v.at[b], agg_sh.at[didx_v.at[j]],
                                 ssem, add=True)
            for j in range(CPB - NBUF, CPB):
                pltpu.make_async_copy(rows_v.at[j % NBUF],
                                      agg_sh.at[didx_v.at[j]], ssem).wait()
            return carry
        lax.fori_loop(0, NBLK, edge_blk, 0)
        plsc.subcore_barrier()

        for ch in range(RPT // RCH):
            rbase = r0 + ch * RCH
            pltpu.sync_copy(g_sh.at[pl.ds(rbase, RCH), :], gbuf_v)
            pltpu.sync_copy(agg_sh.at[pl.ds(rbase, RCH), :], abuf_v)
            for z in range(RCH // ZCH):
                pltpu.sync_copy(zbuf_v,
                                agg_sh.at[pl.ds(rbase + z * ZCH, ZCH), :])
            pltpu.sync_copy(out_hbm.at[c, pl.ds(rbase, RCH), :], hbuf_v)
            def upd_row(r, carry):
                dv2 = _splat(dinv2_v, ch * RCH + r)
                dsq = _splat(dsqrt_v, ch * RCH + r)
                for q in range(HD // 16):
                    sl = pl.ds(q * 16, 16)
                    gn = gbuf_v[r, sl] - abuf_v[r, sl] * dv2
                    hbuf_v[r, sl] = hbuf_v[r, sl] + theta * (gn * dsq)
                    if not last:
                        gbuf_v[r, sl] = gn
                return carry
            lax.fori_loop(0, RCH, upd_row, 0)
            pltpu.sync_copy(hbuf_v, out_hbm.at[c, pl.ds(rbase, RCH), :])
            if not last:
                pltpu.sync_copy(gbuf_v, g_sh.at[pl.ds(rbase, RCH), :])
                pltpu.sync_copy(gbuf_v, g_hbm.at[c, pl.ds(rbase, RCH), :])
        if not last:
            plsc.subcore_barrier()


@jax.jit
def _sc_conv(src, dst, feats):
    mesh = plsc.VectorSubcoreMesh(core_axis_name="c", subcore_axis_name="s")
    return pl.kernel(
        _sc_body,
        out_type=jax.ShapeDtypeStruct((2, NP, HD), jnp.float32),
        mesh=mesh,
        compiler_params=pltpu.CompilerParams(
            needs_layout_passes=False, use_tc_tiling_on_sc=False),
        scratch_types=[
            pltpu.HBM((2, NP, HD), jnp.float32),
            pltpu.VMEM_SHARED((NP, HD), jnp.float32),
            pltpu.VMEM_SHARED((NP, HD), jnp.float32),
            pltpu.VMEM((NBUF, ECH, HD), jnp.float32),
            pltpu.VMEM((ZCH, HD), jnp.float32),
            pltpu.VMEM((RCH, HD), jnp.float32),
            pltpu.VMEM((RCH, HD), jnp.float32),
            pltpu.VMEM((RCH, HD), jnp.float32),
            pltpu.VMEM((CPB, ECH), jnp.int32),
            pltpu.VMEM((CPB, ECH), jnp.int32),
            pltpu.VMEM((RPT,), jnp.float32),
            pltpu.VMEM((RPT,), jnp.float32),
            pltpu.SemaphoreType.DMA,
            pltpu.SemaphoreType.DMA,
            pltpu.SemaphoreType.DMA,
        ],
    )(src, dst, feats)


def kernel(edge_index, feat):
    e0 = edge_index[0]
    e1 = edge_index[1]
    pad = jnp.full((E2P - E2,), N, dtype=jnp.int32)
    src = jnp.concatenate([e0, e1, pad]).reshape(E2P // ECH, ECH)
    dst = jnp.concatenate([e1, e0, pad]).reshape(E2P // ECH, ECH)
    featp = jnp.pad(feat, ((0, NP - N), (0, 0)))
    feats = jnp.stack([featp[:, :HD], featp[:, HD:]], axis=0)
    out = _sc_conv(src, dst, feats)
    return jnp.concatenate([out[0, :N], out[1, :N]], axis=1)


# speedup vs baseline: 1.3100x; 1.3100x over previous
import jax
import jax.numpy as jnp
from jax import lax
from jax.experimental import pallas as pl
from jax.experimental.pallas import tpu as pltpu
from jax.experimental.pallas import tpu_sc as plsc

N = 10000
D = 128
HD = 64
NP = 10240
RPT = 640
RCH = 128
ZCH = 32
NBUF = 2
ECH = 128
CPB = 16
NBLK = 20
EPT = NBLK * CPB * ECH
E2P = 16 * EPT
IPT = EPT // ECH
E2 = 2 * 320000
THETA_K = (-0.5, 0.25, -0.125)
DGR = NP // HD


def _rsqrt(x):
    xi = plsc.bitcast(x, jnp.int32)
    y = plsc.bitcast(jnp.int32(0x5F3759DF) - (xi >> 1), jnp.float32)
    for _ in range(3):
        y = y * (1.5 - 0.5 * x * y * y)
    return y


def _splat(vec_ref, i):
    return plsc.load_gather(vec_ref, [jnp.full((16,), i, jnp.int32)])


def _sc_body(src_hbm, dst_hbm, feat_hbm, out_hbm,
             g_sh, agg_sh,
             rows_v, zbuf_v, gbuf_v, abuf_v, hbuf_v,
             sidx_v, didx_v, dinv2_v, dsqrt_v, gsem, ssem):
    c = lax.axis_index("c")
    s = lax.axis_index("s")
    r0 = s * RPT
    zeros16 = jnp.zeros((16,), jnp.float32)
    ones16 = jnp.ones((16,), jnp.float32)

    def zero_ba(i, carry):
        for q in range(HD // 16):
            gbuf_v[i, pl.ds(q * 16, 16)] = zeros16
            abuf_v[i, pl.ds(q * 16, 16)] = zeros16
        return carry
    lax.fori_loop(0, RCH, zero_ba, 0)

    def deg_blk(blk, carry):
        irow = s * IPT + blk * CPB
        pltpu.sync_copy(dst_hbm.at[pl.ds(irow, CPB), :], didx_v)
        def deg_row(j, carry2):
            def deg_inner(i, carry3):
                idx = didx_v[j, pl.ds(i * 16, 16)]
                row = idx >> 6
                col = idx & 63
                plsc.addupdate_scatter(gbuf_v, [row, col], ones16,
                                       mask=row < RCH)
                plsc.addupdate_scatter(abuf_v, [row - RCH, col], ones16,
                                       mask=row >= RCH)
                return carry3
            return lax.fori_loop(0, ECH // 16, deg_inner, carry2)
        return lax.fori_loop(0, CPB, deg_row, carry)
    lax.fori_loop(0, NBLK, deg_blk, 0)

    pltpu.sync_copy(gbuf_v, agg_sh.at[pl.ds(r0, RCH), :])
    pltpu.sync_copy(abuf_v.at[pl.ds(0, DGR - RCH), :],
                    agg_sh.at[pl.ds(r0 + RCH, DGR - RCH), :])
    plsc.subcore_barrier()

    def zero_g10(i, carry):
        for q in range(HD // 16):
            gbuf_v[i, pl.ds(q * 16, 16)] = zeros16
        return carry
    lax.fori_loop(0, RPT // HD, zero_g10, 0)
    def deg_reduce(t, carry):
        pltpu.sync_copy(agg_sh.at[pl.ds(t * RPT + s * 10, RPT // HD), :],
                        abuf_v.at[pl.ds(0, RPT // HD), :])
        def acc_row(i, carry2):
            for q in range(HD // 16):
                sl = pl.ds(q * 16, 16)
                gbuf_v[i, sl] = gbuf_v[i, sl] + abuf_v[i, sl]
            return carry2
        return lax.fori_loop(0, RPT // HD, acc_row, carry)
    lax.fori_loop(0, 16, deg_reduce, 0)

    def dinv_chunk(j, carry):
        sl = pl.ds((j % 4) * 16, 16)
        x = jnp.maximum(gbuf_v[j // 4, sl], 1.0)
        dv = _rsqrt(x)
        dinv2_v[pl.ds(j * 16, 16)] = dv * dv
        dsqrt_v[pl.ds(j * 16, 16)] = x * dv
        return carry
    lax.fori_loop(0, RPT // 16, dinv_chunk, 0)
    plsc.subcore_barrier()

    def zero_z(i, carry):
        for q in range(HD // 16):
            zbuf_v[i, pl.ds(q * 16, 16)] = zeros16
        return carry
    lax.fori_loop(0, ZCH, zero_z, 0)
    for ch in range(RPT // ZCH):
        pltpu.sync_copy(zbuf_v, agg_sh.at[pl.ds(r0 + ch * ZCH, ZCH), :])
    pltpu.sync_copy(zbuf_v.at[pl.ds(0, 15), :],
                    g_sh.at[pl.ds(N + s * 15, 15), :])

    for ch in range(RPT // RCH):
        rbase = r0 + ch * RCH
        pltpu.sync_copy(feat_hbm.at[c, pl.ds(rbase, RCH), :], gbuf_v)
        pltpu.sync_copy(gbuf_v, out_hbm.at[c, pl.ds(rbase, RCH), :])
        def init_row(r, carry):
            dv = _splat(dinv2_v, ch * RCH + r) * _splat(dsqrt_v, ch * RCH + r)
            for q in range(HD // 16):
                sl = pl.ds(q * 16, 16)
                gbuf_v[r, sl] = gbuf_v[r, sl] * dv
            return carry
        lax.fori_loop(0, RCH, init_row, 0)
        pltpu.sync_copy(gbuf_v, g_sh.at[pl.ds(rbase, RCH), :])

    plsc.subcore_barrier()

    for k, theta in enumerate(THETA_K):
        last = k == len(THETA_K) - 1

        def edge_blk(blk, carry):
            irow = s * IPT + blk * CPB
            pltpu.sync_copy(src_hbm.at[pl.ds(irow, CPB), :], sidx_v)
            pltpu.sync_copy(dst_hbm.at[pl.ds(irow, CPB), :], didx_v)
            for j in range(NBUF - 1):
                pltpu.async_copy(g_sh.at[sidx_v.at[j]], rows_v.at[j], gsem)
            for j in range(CPB):
                b = j % NBUF
                if j + NBUF - 1 < CPB:
                    if j >= 1:
                        pltpu.make_async_copy(
                            rows_v.at[(j - 1) % NBUF],
                            agg_sh.at[didx_v.at[j - 1]], ssem).wait()
                    pltpu.async_copy(
                        g_sh.at[sidx_v.at[j + NBUF - 1]],
                        rows_v.at[(j + NBUF - 1) % NBUF], gsem)
                pltpu.make_async_copy(g_sh.at[sidx_v.at[j]],
                                      rows_v.at[b], gsem).wait()
                pltpu.async_copy(rows_v.at[b], agg_sh.at[didx_v.at[j]],
                                 ssem, add=True)
            for j in range(CPB - NBUF, CPB):
                pltpu.make_async_copy(rows_v.at[j % NBUF],
                                      agg_sh.at[didx_v.at[j]], ssem).wait()
            return carry
        lax.fori_loop(0, NBLK, edge_blk, 0)
        plsc.subcore_barrier()

        for ch in range(RPT // RCH):
            rbase = r0 + ch * RCH
            pltpu.sync_copy(g_sh.at[pl.ds(rbase, RCH), :], gbuf_v)
            pltpu.sync_copy(agg_sh.at[pl.ds(rbase, RCH), :], abuf_v)
            for z in range(RCH // ZCH):
                pltpu.sync_copy(zbuf_v,
                                agg_sh.at[pl.ds(rbase + z * ZCH, ZCH), :])
            pltpu.sync_copy(out_hbm.at[c, pl.ds(rbase, RCH), :], hbuf_v)
            def upd_row(r, carry):
                dv2 = _splat(dinv2_v, ch * RCH + r)
                dsq = _splat(dsqrt_v, ch * RCH + r)
                for q in range(HD // 16):
                    sl = pl.ds(q * 16, 16)
                    gn = gbuf_v[r, sl] - abuf_v[r, sl] * dv2
                    hbuf_v[r, sl] = hbuf_v[r, sl] + theta * (gn * dsq)
                    if not last:
                        gbuf_v[r, sl] = gn
                return carry
            lax.fori_loop(0, RCH, upd_row, 0)
            pltpu.sync_copy(hbuf_v, out_hbm.at[c, pl.ds(rbase, RCH), :])
            if not last:
                pltpu.sync_copy(gbuf_v, g_sh.at[pl.ds(rbase, RCH), :])
        if not last:
            plsc.subcore_barrier()


@jax.jit
def _sc_conv(src, dst, feats):
    mesh = plsc.VectorSubcoreMesh(core_axis_name="c", subcore_axis_name="s")
    return pl.kernel(
        _sc_body,
        out_type=jax.ShapeDtypeStruct((2, NP, HD), jnp.float32),
        mesh=mesh,
        compiler_params=pltpu.CompilerParams(
            needs_layout_passes=False, use_tc_tiling_on_sc=False),
        scratch_types=[
            pltpu.VMEM_SHARED((NP, HD), jnp.float32),
            pltpu.VMEM_SHARED((NP, HD), jnp.float32),
            pltpu.VMEM((NBUF, ECH, HD), jnp.float32),
            pltpu.VMEM((ZCH, HD), jnp.float32),
            pltpu.VMEM((RCH, HD), jnp.float32),
            pltpu.VMEM((RCH, HD), jnp.float32),
            pltpu.VMEM((RCH, HD), jnp.float32),
            pltpu.VMEM((CPB, ECH), jnp.int32),
            pltpu.VMEM((CPB, ECH), jnp.int32),
            pltpu.VMEM((RPT,), jnp.float32),
            pltpu.VMEM((RPT,), jnp.float32),
            pltpu.SemaphoreType.DMA,
            pltpu.SemaphoreType.DMA,
        ],
    )(src, dst, feats)


def kernel(edge_index, feat):
    e0 = edge_index[0]
    e1 = edge_index[1]
    pad = jnp.full((E2P - E2,), N, dtype=jnp.int32)
    src = jnp.concatenate([e0, e1, pad]).reshape(E2P // ECH, ECH)
    dst = jnp.concatenate([e1, e0, pad]).reshape(E2P // ECH, ECH)
    featp = jnp.pad(feat, ((0, NP - N), (0, 0)))
    feats = jnp.stack([featp[:, :HD], featp[:, HD:]], axis=0)
    out = _sc_conv(src, dst, feats)
    return jnp.concatenate([out[0, :N], out[1, :N]], axis=1)
